# baseline (speedup 1.0000x reference)
import functools

import jax
import jax.numpy as jnp
from jax import lax
from jax.experimental import pallas as pl
from jax.experimental.pallas import tpu as pltpu
from jax.experimental.pallas import tpu_sc as plsc

_GAMMA = 1.0
_ALPHA = 0.0001
_BATCH = 4096
_D = 128
_NW = 32
_BPW = _BATCH // _NW
_NBUF = 4


@functools.cache
def _sc_gather_call(tables):
    n = len(tables)
    mesh = plsc.VectorSubcoreMesh(core_axis_name="c", subcore_axis_name="s")

    @functools.partial(
        pl.kernel,
        out_type=jax.ShapeDtypeStruct((n * _BATCH, _D), jnp.float32),
        mesh=mesh,
        scratch_types=(
            [pltpu.VMEM((n, _BPW), jnp.int32),
             pltpu.VMEM((_NBUF, _BPW, _D), jnp.float32),
             pltpu.SemaphoreType.DMA]
            + [pltpu.SemaphoreType.DMA] * (2 * _NBUF)
        ),
    )
    def _sc_gather(*refs):
        idx_refs = refs[:n]
        ent_hbm, rel_hbm, out_hbm, idx_v, rows_v, isem = refs[n:n + 6]
        bufsems = refs[n + 6:]
        gsem = bufsems[:_NBUF]
        wsem = bufsems[_NBUF:]
        wid = lax.axis_index("s") * 2 + lax.axis_index("c")
        base = wid * _BPW

        icopies = [
            pltpu.async_copy(idx_refs[a].at[pl.ds(base, _BPW)], idx_v.at[a],
                             isem)
            for a in range(n)
        ]
        for c in icopies:
            c.wait()

        def gather(a):
            table = rel_hbm if tables[a] else ent_hbm
            b = a % _NBUF
            return pltpu.async_copy(table.at[idx_v.at[a]], rows_v.at[b],
                                    gsem[b])

        g = {}
        w = {}
        for a in range(min(2, n)):
            g[a] = gather(a)
        for a in range(n):
            b = a % _NBUF
            g[a].wait()
            if a + 2 < n:
                if a + 2 - _NBUF >= 0:
                    w[a + 2 - _NBUF].wait()
                g[a + 2] = gather(a + 2)
            w[a] = pltpu.async_copy(
                rows_v.at[b], out_hbm.at[pl.ds(a * _BATCH + base, _BPW)],
                wsem[b])
        for a in range(max(0, n - _NBUF), n):
            w[a].wait()

    return _sc_gather


def _rows(g_ref, a):
    return g_ref[0, a * _BATCH:(a + 1) * _BATCH, :]

def _norm_rows(x):
    return x * lax.rsqrt(jnp.sum(x * x, axis=1, keepdims=True))


def _dist_body(g_ref, o_ref, s1_ref):
    i = pl.program_id(0)
    ones = jnp.ones((_D, _D), jnp.float32)
    ones_v = jnp.ones((_D,), jnp.float32)

    def rowsum_rep(x):
        return lax.dot_general(x, ones, (((1,), (0,)), ((), ())),
                               preferred_element_type=jnp.float32)

    def half_dist():
        x = g_ref[0]
        xn = x * lax.rsqrt(rowsum_rep(x * x))
        d = (xn[0:_BATCH] + xn[2 * _BATCH:3 * _BATCH]
             - xn[_BATCH:2 * _BATCH])
        return jnp.sqrt(rowsum_rep(d * d))

    @pl.when(i == 0)
    def _():
        s1_ref[...] = half_dist()

    @pl.when(i == 1)
    def _():
        o_ref[...] = s1_ref[...] - half_dist()


_dist_call = pl.pallas_call(
    _dist_body,
    grid=(2,),
    in_specs=[pl.BlockSpec((1, 3 * _BATCH, _D), lambda i: (i, 0, 0))],
    out_specs=pl.BlockSpec((_BATCH, _D), lambda i: (0, 0)),
    out_shape=jax.ShapeDtypeStruct((_BATCH, _D), jnp.float32),
    scratch_shapes=[pltpu.VMEM((_BATCH, _D), jnp.float32)],
)


def _reg_body(g_ref, s_ref, o_ref):
    reg = jnp.float32(0.0)
    for a in range(3):
        a_rows = g_ref[a * _BATCH:(a + 1) * _BATCH, :]
        gram = lax.dot_general(a_rows, a_rows, (((0,), (0,)), ((), ())),
                               preferred_element_type=jnp.float32)
        reg = reg + jnp.sqrt(jnp.sum(gram * gram))
    o_ref[...] = _GAMMA + s_ref[:, 0] + _ALPHA * reg


_reg_call = pl.pallas_call(
    _reg_body,
    out_shape=jax.ShapeDtypeStruct((_BATCH,), jnp.float32),
)


def kernel(head, tail, relation, n_head, n_tail, n_relation, reg_user,
           reg_item, reg_brand, entity_embed, relation_embed):
    score_idxs = [x.astype(jnp.int32) for x in (
        head, tail, relation, n_head, n_tail, n_relation)]
    reg_idxs = [x.astype(jnp.int32) for x in (reg_user, reg_item, reg_brand)]
    score_rows = _sc_gather_call((0, 0, 1, 0, 0, 1))(
        *score_idxs, entity_embed, relation_embed)
    reg_rows = _sc_gather_call((0, 0, 0))(
        *reg_idxs, entity_embed, relation_embed)
    sdiff = _dist_call(score_rows.reshape(2, 3 * _BATCH, _D))
    return _reg_call(reg_rows, sdiff)


# speedup vs baseline: 1.0868x; 1.0868x over previous
import functools

import jax
import jax.numpy as jnp
from jax import lax
from jax.experimental import pallas as pl
from jax.experimental.pallas import tpu as pltpu
from jax.experimental.pallas import tpu_sc as plsc

_GAMMA = 1.0
_ALPHA = 0.0001
_BATCH = 4096
_D = 128
_NW = 32
_BPW = _BATCH // _NW
_NBUF = 4


@functools.cache
def _sc_gather_call(tables):
    n = len(tables)
    mesh = plsc.VectorSubcoreMesh(core_axis_name="c", subcore_axis_name="s")

    @functools.partial(
        pl.kernel,
        out_type=jax.ShapeDtypeStruct((n * _BATCH, _D), jnp.float32),
        mesh=mesh,
        scratch_types=(
            [pltpu.VMEM((n, _BPW), jnp.int32),
             pltpu.VMEM((_NBUF, _BPW, _D), jnp.float32),
             pltpu.SemaphoreType.DMA]
            + [pltpu.SemaphoreType.DMA] * (2 * _NBUF)
        ),
    )
    def _sc_gather(*refs):
        idx_refs = refs[:n]
        ent_hbm, rel_hbm, out_hbm, idx_v, rows_v, isem = refs[n:n + 6]
        bufsems = refs[n + 6:]
        gsem = bufsems[:_NBUF]
        wsem = bufsems[_NBUF:]
        wid = lax.axis_index("s") * 2 + lax.axis_index("c")
        base = wid * _BPW

        icopies = [
            pltpu.async_copy(idx_refs[a].at[pl.ds(base, _BPW)], idx_v.at[a],
                             isem)
            for a in range(n)
        ]
        for c in icopies:
            c.wait()

        def gather(a):
            table = rel_hbm if tables[a] else ent_hbm
            b = a % _NBUF
            return pltpu.async_copy(table.at[idx_v.at[a]], rows_v.at[b],
                                    gsem[b])

        depth = 3
        g = {}
        w = {}
        for a in range(min(depth, n)):
            g[a] = gather(a)
        for a in range(n):
            b = a % _NBUF
            g[a].wait()
            if a + depth < n:
                if a + depth - _NBUF >= 0:
                    w[a + depth - _NBUF].wait()
                g[a + depth] = gather(a + depth)
            w[a] = pltpu.async_copy(
                rows_v.at[b], out_hbm.at[pl.ds(a * _BATCH + base, _BPW)],
                wsem[b])
        for a in range(max(0, n - _NBUF), n):
            w[a].wait()

    return _sc_gather


def _rows(g_ref, a):
    return g_ref[0, a * _BATCH:(a + 1) * _BATCH, :]

def _norm_rows(x):
    return x * lax.rsqrt(jnp.sum(x * x, axis=1, keepdims=True))


def _dist_body(g_ref, o_ref, s1_ref):
    i = pl.program_id(0)
    ones = jnp.ones((_D, _D), jnp.float32)
    ones_v = jnp.ones((_D,), jnp.float32)

    def rowsum_rep(x):
        return lax.dot_general(x, ones, (((1,), (0,)), ((), ())),
                               preferred_element_type=jnp.float32)

    def half_dist():
        x = g_ref[0]
        xn = x * lax.rsqrt(rowsum_rep(x * x))
        d = (xn[0:_BATCH] + xn[2 * _BATCH:3 * _BATCH]
             - xn[_BATCH:2 * _BATCH])
        return jnp.sqrt(rowsum_rep(d * d))

    @pl.when(i == 0)
    def _():
        s1_ref[...] = half_dist()

    @pl.when(i == 1)
    def _():
        o_ref[...] = (s1_ref[...] - half_dist())[:, 0]


_dist_call = pl.pallas_call(
    _dist_body,
    grid=(2,),
    in_specs=[pl.BlockSpec((1, 3 * _BATCH, _D), lambda i: (i, 0, 0))],
    out_specs=pl.BlockSpec((_BATCH,), lambda i: (0,)),
    out_shape=jax.ShapeDtypeStruct((_BATCH,), jnp.float32),
    scratch_shapes=[pltpu.VMEM((_BATCH, _D), jnp.float32)],
)


def _reg_body(g_ref, s_ref, o_ref):
    reg = jnp.float32(0.0)
    for a in range(3):
        a_rows = g_ref[a * _BATCH:(a + 1) * _BATCH, :]
        gram = lax.dot_general(a_rows, a_rows, (((0,), (0,)), ((), ())),
                               preferred_element_type=jnp.float32)
        reg = reg + jnp.sqrt(jnp.sum(gram * gram))
    o_ref[...] = _GAMMA + s_ref[...] + _ALPHA * reg


_reg_call = pl.pallas_call(
    _reg_body,
    out_shape=jax.ShapeDtypeStruct((_BATCH,), jnp.float32),
)


def kernel(head, tail, relation, n_head, n_tail, n_relation, reg_user,
           reg_item, reg_brand, entity_embed, relation_embed):
    score_idxs = [x.astype(jnp.int32) for x in (
        head, tail, relation, n_head, n_tail, n_relation)]
    reg_idxs = [x.astype(jnp.int32) for x in (reg_user, reg_item, reg_brand)]
    score_rows = _sc_gather_call((0, 0, 1, 0, 0, 1))(
        *score_idxs, entity_embed, relation_embed)
    reg_rows = _sc_gather_call((0, 0, 0))(
        *reg_idxs, entity_embed, relation_embed)
    sdiff = _dist_call(score_rows.reshape(2, 3 * _BATCH, _D))
    return _reg_call(reg_rows, sdiff)


# speedup vs baseline: 1.0892x; 1.0022x over previous
import functools

import jax
import jax.numpy as jnp
from jax import lax
from jax.experimental import pallas as pl
from jax.experimental.pallas import tpu as pltpu
from jax.experimental.pallas import tpu_sc as plsc

_GAMMA = 1.0
_ALPHA = 0.0001
_BATCH = 4096
_D = 128
_NW = 32
_BPW = _BATCH // _NW
_NBUF = 6


@functools.cache
def _sc_gather_call(tables):
    n = len(tables)
    mesh = plsc.VectorSubcoreMesh(core_axis_name="c", subcore_axis_name="s")

    @functools.partial(
        pl.kernel,
        out_type=jax.ShapeDtypeStruct((n * _BATCH, _D), jnp.float32),
        mesh=mesh,
        scratch_types=(
            [pltpu.VMEM((n, _BPW), jnp.int32),
             pltpu.VMEM((_NBUF, _BPW, _D), jnp.float32),
             pltpu.SemaphoreType.DMA]
            + [pltpu.SemaphoreType.DMA] * (2 * _NBUF)
        ),
    )
    def _sc_gather(*refs):
        idx_refs = refs[:n]
        ent_hbm, rel_hbm, out_hbm, idx_v, rows_v, isem = refs[n:n + 6]
        bufsems = refs[n + 6:]
        gsem = bufsems[:_NBUF]
        wsem = bufsems[_NBUF:]
        wid = lax.axis_index("s") * 2 + lax.axis_index("c")
        base = wid * _BPW

        icopies = [
            pltpu.async_copy(idx_refs[a].at[pl.ds(base, _BPW)], idx_v.at[a],
                             isem)
            for a in range(n)
        ]
        for c in icopies:
            c.wait()

        def gather(a):
            table = rel_hbm if tables[a] else ent_hbm
            b = a % _NBUF
            return pltpu.async_copy(table.at[idx_v.at[a]], rows_v.at[b],
                                    gsem[b])

        depth = 4
        g = {}
        w = {}
        for a in range(min(depth, n)):
            g[a] = gather(a)
        for a in range(n):
            b = a % _NBUF
            g[a].wait()
            if a + depth < n:
                if a + depth - _NBUF >= 0:
                    w[a + depth - _NBUF].wait()
                g[a + depth] = gather(a + depth)
            w[a] = pltpu.async_copy(
                rows_v.at[b], out_hbm.at[pl.ds(a * _BATCH + base, _BPW)],
                wsem[b])
        for a in range(max(0, n - _NBUF), n):
            w[a].wait()

    return _sc_gather


def _rows(g_ref, a):
    return g_ref[0, a * _BATCH:(a + 1) * _BATCH, :]

def _norm_rows(x):
    return x * lax.rsqrt(jnp.sum(x * x, axis=1, keepdims=True))


def _dist_body(g_ref, o_ref, s1_ref):
    i = pl.program_id(0)
    ones = jnp.ones((_D, _D), jnp.float32)
    ones_v = jnp.ones((_D,), jnp.float32)

    def rowsum_rep(x):
        return lax.dot_general(x, ones, (((1,), (0,)), ((), ())),
                               preferred_element_type=jnp.float32)

    def half_dist():
        x = g_ref[0]
        xn = x * lax.rsqrt(rowsum_rep(x * x))
        d = (xn[0:_BATCH] + xn[2 * _BATCH:3 * _BATCH]
             - xn[_BATCH:2 * _BATCH])
        return jnp.sqrt(rowsum_rep(d * d))

    @pl.when(i == 0)
    def _():
        s1_ref[...] = half_dist()

    @pl.when(i == 1)
    def _():
        o_ref[...] = (s1_ref[...] - half_dist())[:, 0]


_dist_call = pl.pallas_call(
    _dist_body,
    grid=(2,),
    in_specs=[pl.BlockSpec((1, 3 * _BATCH, _D), lambda i: (i, 0, 0))],
    out_specs=pl.BlockSpec((_BATCH,), lambda i: (0,)),
    out_shape=jax.ShapeDtypeStruct((_BATCH,), jnp.float32),
    scratch_shapes=[pltpu.VMEM((_BATCH, _D), jnp.float32)],
)


def _reg_body(g_ref, s_ref, o_ref, acc_ref):
    i = pl.program_id(0)
    a_rows = g_ref[0]
    gram = lax.dot_general(a_rows, a_rows, (((0,), (0,)), ((), ())),
                           preferred_element_type=jnp.float32)
    part = jnp.sqrt(jnp.sum(gram * gram))

    @pl.when(i == 0)
    def _():
        acc_ref[0] = part

    @pl.when(i == 1)
    def _():
        acc_ref[0] = acc_ref[0] + part

    @pl.when(i == 2)
    def _():
        o_ref[...] = _GAMMA + s_ref[...] + _ALPHA * (acc_ref[0] + part)


_reg_call = pl.pallas_call(
    _reg_body,
    grid=(3,),
    in_specs=[pl.BlockSpec((1, _BATCH, _D), lambda i: (i, 0, 0)),
              pl.BlockSpec((_BATCH,), lambda i: (0,))],
    out_specs=pl.BlockSpec((_BATCH,), lambda i: (0,)),
    out_shape=jax.ShapeDtypeStruct((_BATCH,), jnp.float32),
    scratch_shapes=[pltpu.SMEM((1,), jnp.float32)],
)


def kernel(head, tail, relation, n_head, n_tail, n_relation, reg_user,
           reg_item, reg_brand, entity_embed, relation_embed):
    score_idxs = [x.astype(jnp.int32) for x in (
        head, tail, relation, n_head, n_tail, n_relation)]
    reg_idxs = [x.astype(jnp.int32) for x in (reg_user, reg_item, reg_brand)]
    score_rows = _sc_gather_call((0, 0, 1, 0, 0, 1))(
        *score_idxs, entity_embed, relation_embed)
    reg_rows = _sc_gather_call((0, 0, 0))(
        *reg_idxs, entity_embed, relation_embed)
    sdiff = _dist_call(score_rows.reshape(2, 3 * _BATCH, _D))
    return _reg_call(reg_rows.reshape(3, _BATCH, _D), sdiff)


# speedup vs baseline: 1.0922x; 1.0028x over previous
import functools

import jax
import jax.numpy as jnp
from jax import lax
from jax.experimental import pallas as pl
from jax.experimental.pallas import tpu as pltpu
from jax.experimental.pallas import tpu_sc as plsc

_GAMMA = 1.0
_ALPHA = 0.0001
_BATCH = 4096
_D = 128
_NW = 32
_BPW = _BATCH // _NW
_NBUF = 6


@functools.cache
def _sc_gather_call(tables):
    n = len(tables)
    mesh = plsc.VectorSubcoreMesh(core_axis_name="c", subcore_axis_name="s")

    @functools.partial(
        pl.kernel,
        out_type=jax.ShapeDtypeStruct((n * _BATCH, _D), jnp.float32),
        mesh=mesh,
        scratch_types=(
            [pltpu.VMEM((n, _BPW), jnp.int32),
             pltpu.VMEM((_NBUF, _BPW, _D), jnp.float32),
             pltpu.SemaphoreType.DMA]
            + [pltpu.SemaphoreType.DMA] * (2 * _NBUF)
        ),
    )
    def _sc_gather(*refs):
        idx_refs = refs[:n]
        ent_hbm, rel_hbm, out_hbm, idx_v, rows_v, isem = refs[n:n + 6]
        bufsems = refs[n + 6:]
        gsem = bufsems[:_NBUF]
        wsem = bufsems[_NBUF:]
        wid = lax.axis_index("s") * 2 + lax.axis_index("c")
        base = wid * _BPW

        icopies = [
            pltpu.async_copy(idx_refs[a].at[pl.ds(base, _BPW)], idx_v.at[a],
                             isem)
            for a in range(n)
        ]
        for c in icopies:
            c.wait()

        def gather(a):
            table = rel_hbm if tables[a] else ent_hbm
            b = a % _NBUF
            return pltpu.async_copy(table.at[idx_v.at[a]], rows_v.at[b],
                                    gsem[b])

        depth = 4
        g = {}
        w = {}
        for a in range(min(depth, n)):
            g[a] = gather(a)
        for a in range(n):
            b = a % _NBUF
            g[a].wait()
            if a + depth < n:
                if a + depth - _NBUF >= 0:
                    w[a + depth - _NBUF].wait()
                g[a + depth] = gather(a + depth)
            w[a] = pltpu.async_copy(
                rows_v.at[b], out_hbm.at[pl.ds(a * _BATCH + base, _BPW)],
                wsem[b])
        for a in range(max(0, n - _NBUF), n):
            w[a].wait()

    return _sc_gather


def _rows(g_ref, a):
    return g_ref[0, a * _BATCH:(a + 1) * _BATCH, :]

def _norm_rows(x):
    return x * lax.rsqrt(jnp.sum(x * x, axis=1, keepdims=True))


def _dist_body(ht_ref, r_ref, o_ref, s1_ref):
    i = pl.program_id(0)
    ones = jnp.ones((_D, _D), jnp.float32)
    ones_v = jnp.ones((_D,), jnp.float32)

    def rowsum_rep(x):
        return lax.dot_general(x, ones, (((1,), (0,)), ((), ())),
                               preferred_element_type=jnp.float32)

    def half_dist():
        ht = ht_ref[0]
        r = r_ref[0]
        htn = ht * lax.rsqrt(rowsum_rep(ht * ht))
        rn = r * lax.rsqrt(rowsum_rep(r * r))
        d = htn[0:_BATCH] + rn - htn[_BATCH:2 * _BATCH]
        return jnp.sqrt(rowsum_rep(d * d))

    @pl.when(i == 0)
    def _():
        s1_ref[...] = half_dist()

    @pl.when(i == 1)
    def _():
        o_ref[...] = (s1_ref[...] - half_dist())[:, 0]


_dist_call = pl.pallas_call(
    _dist_body,
    grid=(2,),
    in_specs=[pl.BlockSpec((1, 2 * _BATCH, _D), lambda i: (i, 0, 0)),
              pl.BlockSpec((1, _BATCH, _D), lambda i: (i, 2, 0))],
    out_specs=pl.BlockSpec((_BATCH,), lambda i: (0,)),
    out_shape=jax.ShapeDtypeStruct((_BATCH,), jnp.float32),
    scratch_shapes=[pltpu.VMEM((_BATCH, _D), jnp.float32)],
)


def _reg_body(u_ref, i_ref, b_ref, s_ref, o_ref):
    reg = jnp.float32(0.0)
    for ref in (u_ref, i_ref, b_ref):
        a_rows = ref[0]
        gram = lax.dot_general(a_rows, a_rows, (((0,), (0,)), ((), ())),
                               preferred_element_type=jnp.float32)
        reg = reg + jnp.sqrt(jnp.sum(gram * gram))
    o_ref[...] = _GAMMA + s_ref[...] + _ALPHA * reg


_reg_call = pl.pallas_call(
    _reg_body,
    grid=(1,),
    in_specs=[pl.BlockSpec((1, _BATCH, _D), lambda i: (0, 0, 0)),
              pl.BlockSpec((1, _BATCH, _D), lambda i: (1, 0, 0)),
              pl.BlockSpec((1, _BATCH, _D), lambda i: (2, 0, 0)),
              pl.BlockSpec((_BATCH,), lambda i: (0,))],
    out_specs=pl.BlockSpec((_BATCH,), lambda i: (0,)),
    out_shape=jax.ShapeDtypeStruct((_BATCH,), jnp.float32),
)


def kernel(head, tail, relation, n_head, n_tail, n_relation, reg_user,
           reg_item, reg_brand, entity_embed, relation_embed):
    score_idxs = [x.astype(jnp.int32) for x in (
        head, tail, relation, n_head, n_tail, n_relation)]
    reg_idxs = [x.astype(jnp.int32) for x in (reg_user, reg_item, reg_brand)]
    score_rows = _sc_gather_call((0, 0, 1, 0, 0, 1))(
        *score_idxs, entity_embed, relation_embed)
    reg_rows = _sc_gather_call((0, 0, 0))(
        *reg_idxs, entity_embed, relation_embed)
    score3d = score_rows.reshape(2, 3 * _BATCH, _D)
    reg3d = reg_rows.reshape(3, _BATCH, _D)
    sdiff = _dist_call(score3d, score3d)
    return _reg_call(reg3d, reg3d, reg3d, sdiff)


# speedup vs baseline: 1.1151x; 1.0210x over previous
import functools

import jax
import jax.numpy as jnp
from jax import lax
from jax.experimental import pallas as pl
from jax.experimental.pallas import tpu as pltpu
from jax.experimental.pallas import tpu_sc as plsc

_GAMMA = 1.0
_ALPHA = 0.0001
_BATCH = 4096
_D = 128
_NW = 32
_BPW = _BATCH // _NW
_NBUF = 6


@functools.cache
def _sc_gather_call(tables):
    n = len(tables)
    mesh = plsc.VectorSubcoreMesh(core_axis_name="c", subcore_axis_name="s")

    @functools.partial(
        pl.kernel,
        out_type=jax.ShapeDtypeStruct((n * _BATCH, _D), jnp.float32),
        mesh=mesh,
        scratch_types=(
            [pltpu.VMEM((n, _BPW), jnp.int32),
             pltpu.VMEM((_NBUF, _BPW, _D), jnp.float32),
             pltpu.SemaphoreType.DMA]
            + [pltpu.SemaphoreType.DMA] * (2 * _NBUF)
        ),
    )
    def _sc_gather(*refs):
        idx_refs = refs[:n]
        ent_hbm, rel_hbm, out_hbm, idx_v, rows_v, isem = refs[n:n + 6]
        bufsems = refs[n + 6:]
        gsem = bufsems[:_NBUF]
        wsem = bufsems[_NBUF:]
        wid = lax.axis_index("s") * 2 + lax.axis_index("c")
        base = wid * _BPW

        icopies = [
            pltpu.async_copy(idx_refs[a].at[pl.ds(base, _BPW)], idx_v.at[a],
                             isem)
            for a in range(n)
        ]
        for c in icopies:
            c.wait()

        def gather(a):
            table = rel_hbm if tables[a] else ent_hbm
            b = a % _NBUF
            return pltpu.async_copy(table.at[idx_v.at[a]], rows_v.at[b],
                                    gsem[b])

        depth = 6
        g = {}
        w = {}
        for a in range(min(depth, n)):
            g[a] = gather(a)
        for a in range(n):
            b = a % _NBUF
            g[a].wait()
            if a + depth < n:
                if a + depth - _NBUF >= 0:
                    w[a + depth - _NBUF].wait()
                g[a + depth] = gather(a + depth)
            w[a] = pltpu.async_copy(
                rows_v.at[b], out_hbm.at[pl.ds(a * _BATCH + base, _BPW)],
                wsem[b])
        for a in range(max(0, n - _NBUF), n):
            w[a].wait()

    return _sc_gather


def _rows(g_ref, a):
    return g_ref[0, a * _BATCH:(a + 1) * _BATCH, :]

def _norm_rows(x):
    return x * lax.rsqrt(jnp.sum(x * x, axis=1, keepdims=True))


def _dist_body(ht_ref, r_ref, o_ref, s1_ref):
    i = pl.program_id(0)
    ones = jnp.ones((_D, _D), jnp.float32)
    ones_v = jnp.ones((_D,), jnp.float32)

    def rowsum_rep(x):
        return lax.dot_general(x, ones, (((1,), (0,)), ((), ())),
                               preferred_element_type=jnp.float32)

    def half_dist():
        ht = ht_ref[0]
        r = r_ref[0]
        htn = ht * lax.rsqrt(rowsum_rep(ht * ht))
        rn = r * lax.rsqrt(rowsum_rep(r * r))
        d = htn[0:_BATCH] + rn - htn[_BATCH:2 * _BATCH]
        return jnp.sqrt(rowsum_rep(d * d))

    @pl.when(i == 0)
    def _():
        s1_ref[...] = half_dist()

    @pl.when(i == 1)
    def _():
        o_ref[...] = (s1_ref[...] - half_dist())[:, 0]


_dist_call = pl.pallas_call(
    _dist_body,
    grid=(2,),
    in_specs=[pl.BlockSpec((1, 2 * _BATCH, _D), lambda i: (i, 0, 0)),
              pl.BlockSpec((1, _BATCH, _D), lambda i: (i, 2, 0))],
    out_specs=pl.BlockSpec((_BATCH,), lambda i: (0,)),
    out_shape=jax.ShapeDtypeStruct((_BATCH,), jnp.float32),
    scratch_shapes=[pltpu.VMEM((_BATCH, _D), jnp.float32)],
)


def _reg_body(u_ref, i_ref, b_ref, s_ref, o_ref):
    reg = jnp.float32(0.0)
    for ref in (u_ref, i_ref, b_ref):
        a_rows = ref[0]
        gram = lax.dot_general(a_rows, a_rows, (((0,), (0,)), ((), ())),
                               preferred_element_type=jnp.float32)
        reg = reg + jnp.sqrt(jnp.sum(gram * gram))
    o_ref[...] = _GAMMA + s_ref[...] + _ALPHA * reg


_reg_call = pl.pallas_call(
    _reg_body,
    grid=(1,),
    in_specs=[pl.BlockSpec((1, _BATCH, _D), lambda i: (0, 0, 0)),
              pl.BlockSpec((1, _BATCH, _D), lambda i: (1, 0, 0)),
              pl.BlockSpec((1, _BATCH, _D), lambda i: (2, 0, 0)),
              pl.BlockSpec((_BATCH,), lambda i: (0,))],
    out_specs=pl.BlockSpec((_BATCH,), lambda i: (0,)),
    out_shape=jax.ShapeDtypeStruct((_BATCH,), jnp.float32),
)


def kernel(head, tail, relation, n_head, n_tail, n_relation, reg_user,
           reg_item, reg_brand, entity_embed, relation_embed):
    score_idxs = [x.astype(jnp.int32) for x in (
        head, tail, relation, n_head, n_tail, n_relation)]
    reg_idxs = [x.astype(jnp.int32) for x in (reg_user, reg_item, reg_brand)]
    score_rows = _sc_gather_call((0, 0, 1, 0, 0, 1))(
        *score_idxs, entity_embed, relation_embed)
    reg_rows = _sc_gather_call((0, 0, 0))(
        *reg_idxs, entity_embed, relation_embed)
    score3d = score_rows.reshape(2, 3 * _BATCH, _D)
    reg3d = reg_rows.reshape(3, _BATCH, _D)
    sdiff = _dist_call(score3d, score3d)
    return _reg_call(reg3d, reg3d, reg3d, sdiff)
